# speedup vs baseline: 27.2737x; 27.2737x over previous
import jax
import jax.numpy as jnp
from jax import lax
from jax.experimental import pallas as pl
from jax.experimental.pallas import tpu as pltpu
from jax.experimental.pallas import tpu_sc as plsc

NUM_NODES = 10000
NUM_EDGES = 320000
D = 128
NC = 2
NS = 16
NW = NC * NS
HC = 80
EPT = NUM_EDGES // NW
CPT = EPT // HC
NSL = NUM_NODES // HC
EPH = NUM_EDGES // NS
CPH = EPH // HC

_LANES = 16
_mesh = plsc.VectorSubcoreMesh(core_axis_name="c", subcore_axis_name="s")


def _fill_f32(ref, n, value):
    v = jnp.full((_LANES,), value, jnp.float32)
    for k in range(n // _LANES):
        ref[pl.ds(k * _LANES, _LANES)] = v


def _hist_body(ep_hbm, deg_hbm, ibuf, vbuf, hist, _sem):
    c = lax.axis_index("c")
    s = lax.axis_index("s")
    _fill_f32(vbuf, HC, 0.0)
    for t in range(8):
        j = t * NS + s

        @pl.when(j < NSL)
        def _():
            pltpu.sync_copy(vbuf, hist.at[pl.ds(j * HC, HC)])

    plsc.subcore_barrier()
    _fill_f32(vbuf, HC, 1.0)

    def chunk(j, carry):
        base = c * NUM_EDGES + s * EPH + j * HC
        pltpu.sync_copy(ep_hbm.at[pl.ds(base, HC)], ibuf)
        pltpu.sync_copy(vbuf, hist.at[ibuf], add=True)
        return carry

    lax.fori_loop(0, CPH, chunk, 0)
    plsc.subcore_barrier()

    @pl.when(s == 0)
    def _():
        pltpu.sync_copy(hist, deg_hbm.at[c])


def _prop_body(y_hbm, src_hbm, dst_hbm, p_hbm, sib, dib, ra, rb, acc,
               sem_a, sem_b):
    c = lax.axis_index("c")
    s = lax.axis_index("s")
    wid = c * NS + s

    def zrow(i, carry):
        for k in range(D // _LANES):
            ra[i, pl.ds(k * _LANES, _LANES)] = jnp.zeros((_LANES,),
                                                         jnp.float32)
        return carry

    lax.fori_loop(0, HC, zrow, 0)
    for t in range(8):
        j = t * NS + s

        @pl.when(j < NSL)
        def _():
            pltpu.sync_copy(ra, acc.at[pl.ds(j * HC, HC)])

    plsc.subcore_barrier()

    def chunk(j, carry):
        base = wid * EPT + j * HC
        pltpu.sync_copy(src_hbm.at[pl.ds(base, HC)], sib)
        pltpu.sync_copy(dst_hbm.at[pl.ds(base, HC)], dib)
        pltpu.async_copy(y_hbm.at[dib], ra, sem_a).wait()
        pltpu.sync_copy(ra, acc.at[sib], add=True)
        pltpu.async_copy(y_hbm.at[sib], rb, sem_b).wait()
        pltpu.sync_copy(rb, acc.at[dib], add=True)
        return carry

    lax.fori_loop(0, CPT, chunk, 0)
    plsc.subcore_barrier()
    for t in range(8):
        j = t * NS + s

        @pl.when(j < NSL)
        def _():
            pltpu.sync_copy(acc.at[pl.ds(j * HC, HC)],
                            p_hbm.at[c, pl.ds(j * HC, HC)])


def _scale_body(degt_ref, x_ref, y_ref):
    d = degt_ref[:, 0:1] + degt_ref[:, 1:2]
    r = lax.rsqrt(jnp.maximum(d, 1.0))
    y_ref[...] = r * x_ref[...]


def _final_body(degt_ref, p_ref, o_ref):
    d = degt_ref[:, 0:1] + degt_ref[:, 1:2]
    r = lax.rsqrt(jnp.maximum(d, 1.0))
    o_ref[...] = r * (p_ref[0] + p_ref[1])


_hist = pl.kernel(
    _hist_body,
    out_type=jax.ShapeDtypeStruct((NC, NUM_NODES), jnp.float32),
    mesh=_mesh,
    scratch_types=[
        pltpu.VMEM((HC,), jnp.int32),
        pltpu.VMEM((HC,), jnp.float32),
        pltpu.VMEM_SHARED((NUM_NODES,), jnp.float32),
        pltpu.SemaphoreType.DMA,
    ],
)

_prop = pl.kernel(
    _prop_body,
    out_type=jax.ShapeDtypeStruct((NC, NUM_NODES, D), jnp.float32),
    mesh=_mesh,
    scratch_types=[
        pltpu.VMEM((HC,), jnp.int32),
        pltpu.VMEM((HC,), jnp.int32),
        pltpu.VMEM((HC, D), jnp.float32),
        pltpu.VMEM((HC, D), jnp.float32),
        pltpu.VMEM_SHARED((NUM_NODES, D), jnp.float32),
        pltpu.SemaphoreType.DMA,
        pltpu.SemaphoreType.DMA,
    ],
)

_scale = pl.pallas_call(
    _scale_body,
    out_shape=jax.ShapeDtypeStruct((NUM_NODES, D), jnp.float32),
)

_final = pl.pallas_call(
    _final_body,
    out_shape=jax.ShapeDtypeStruct((NUM_NODES, D), jnp.float32),
)


@jax.jit
def kernel(x, edge_index):
    ei = edge_index.astype(jnp.int32)
    src = ei[0]
    dst = ei[1]
    ep = ei.reshape(-1)
    deg_part = _hist(ep)
    degt = deg_part.T
    y = _scale(degt, x)
    p = _prop(y, src, dst)
    return _final(degt, p)


# speedup vs baseline: 44.1861x; 1.6201x over previous
import jax
import jax.numpy as jnp
from jax import lax
from jax.experimental import pallas as pl
from jax.experimental.pallas import tpu as pltpu
from jax.experimental.pallas import tpu_sc as plsc

NUM_NODES = 10000
NUM_EDGES = 320000
D = 128
NC = 2
NS = 16
NW = NC * NS
HC = 40
EPT = NUM_EDGES // NW
CPT = EPT // HC
NB = 2
GRP2 = (CPT - NB) // (2 * NB)
NSL = NUM_NODES // HC
HCH = 80
CPH = NUM_EDGES // NS // HCH

_LANES = 16
_mesh = plsc.VectorSubcoreMesh(core_axis_name="c", subcore_axis_name="s")


def _fill_f32(ref, n, value):
    v = jnp.full((_LANES,), value, jnp.float32)
    for k in range(n // _LANES):
        ref[pl.ds(k * _LANES, _LANES)] = v


def _hist_body(ep_hbm, deg_hbm, idx2, vbuf, hist, sem):
    c = lax.axis_index("c")
    s = lax.axis_index("s")
    _fill_f32(vbuf, HCH, 0.0)
    for t in range(8):
        j = t * NS + s

        @pl.when(j < NUM_NODES // HCH)
        def _():
            pltpu.sync_copy(vbuf, hist.at[pl.ds(j * HCH, HCH)])

    plsc.subcore_barrier()
    _fill_f32(vbuf, HCH, 1.0)
    row0 = (c * NS + s) * CPH
    pltpu.sync_copy(ep_hbm.at[pl.ds(row0, CPH)], idx2)

    def fire(j, carry):
        pltpu.async_copy(vbuf, hist.at[idx2.at[j, 0]], sem, add=True)
        return carry

    lax.fori_loop(0, CPH, fire, 0)

    def drain(j, carry):
        pltpu.make_async_copy(vbuf, hist.at[idx2.at[j, 0]], sem).wait()
        return carry

    lax.fori_loop(0, CPH, drain, 0)
    plsc.subcore_barrier()

    @pl.when(s == 0)
    def _():
        pltpu.sync_copy(hist, deg_hbm.at[c])


def _prop_body(y_hbm, src_hbm, dst_hbm, p_hbm,
               sib_a, dib_a, sib_b, dib_b,
               a0, a1, a2, a3, b0, b1, b2, b3,
               acc, ss_a, ss_b,
               ga0, ga1, ga2, ga3, gb0, gb1, gb2, gb3):
    c = lax.axis_index("c")
    s = lax.axis_index("s")
    wid = c * NS + s
    sets = (
        (sib_a, dib_a, (a0, a1, a2, a3), ss_a, (ga0, ga1, ga2, ga3)),
        (sib_b, dib_b, (b0, b1, b2, b3), ss_b, (gb0, gb1, gb2, gb3)),
    )

    def zrow(i, carry):
        for k in range(D // _LANES):
            a0[i, pl.ds(k * _LANES, _LANES)] = jnp.zeros((_LANES,),
                                                         jnp.float32)
        return carry

    lax.fori_loop(0, HC, zrow, 0)
    for t in range(16):
        j = t * NS + s

        @pl.when(j < NSL)
        def _():
            pltpu.sync_copy(a0, acc.at[pl.ds(j * HC, HC)])

    plsc.subcore_barrier()

    def unit_refs(st, u):
        sib, dib, bufs, ss, gs = st
        k = u // 2
        if u % 2 == 0:
            gi, si = dib.at[k, 0], sib.at[k, 0]
        else:
            gi, si = sib.at[k, 0], dib.at[k, 0]
        return gi, si, bufs[u], ss, gs[u]

    def drain_set(st):
        for u in range(2 * NB):
            gi, si, buf, ss, _ = unit_refs(st, u)
            pltpu.make_async_copy(buf, acc.at[si], ss).wait()

    def phase(st, j0, first):
        sib, dib, bufs, ss, gs = st
        if not first:
            drain_set(st)
        pltpu.sync_copy(src_hbm.at[pl.ds(wid * CPT + j0, NB)], sib)
        pltpu.sync_copy(dst_hbm.at[pl.ds(wid * CPT + j0, NB)], dib)
        for u in range(2 * NB):
            gi, si, buf, ss_, gsem = unit_refs(st, u)
            pltpu.async_copy(y_hbm.at[gi], buf, gsem)
        for u in range(2 * NB):
            gi, si, buf, ss_, gsem = unit_refs(st, u)
            pltpu.make_async_copy(y_hbm.at[gi], buf, gsem).wait()
            pltpu.async_copy(buf, acc.at[si], ss_, add=True)

    def pair(h, carry):
        @pl.when(h == 0)
        def _():
            phase(sets[0], h * 2 * NB, True)
            phase(sets[1], h * 2 * NB + NB, True)

        @pl.when(h > 0)
        def _():
            phase(sets[0], h * 2 * NB, False)
            phase(sets[1], h * 2 * NB + NB, False)

        return carry

    lax.fori_loop(0, GRP2, pair, 0)
    phase(sets[0], 2 * NB * GRP2, False)
    drain_set(sets[0])
    drain_set(sets[1])
    plsc.subcore_barrier()
    for t in range(16):
        j = t * NS + s

        @pl.when(j < NSL)
        def _():
            pltpu.sync_copy(acc.at[pl.ds(j * HC, HC)],
                            p_hbm.at[c, pl.ds(j * HC, HC)])


def _scale_body(degt_ref, x_ref, y_ref):
    d = degt_ref[:, 0:1] + degt_ref[:, 1:2]
    r = lax.rsqrt(jnp.maximum(d, 1.0))
    y_ref[...] = r * x_ref[...]


def _final_body(degt_ref, p_ref, o_ref):
    d = degt_ref[:, 0:1] + degt_ref[:, 1:2]
    r = lax.rsqrt(jnp.maximum(d, 1.0))
    o_ref[...] = r * (p_ref[0] + p_ref[1])


_hist = pl.kernel(
    _hist_body,
    out_type=jax.ShapeDtypeStruct((NC, NUM_NODES), jnp.float32),
    mesh=_mesh,
    scratch_types=[
        pltpu.VMEM((CPH, 1, HCH), jnp.int32),
        pltpu.VMEM((HCH,), jnp.float32),
        pltpu.VMEM_SHARED((NUM_NODES,), jnp.float32),
        pltpu.SemaphoreType.DMA,
    ],
)

_prop = pl.kernel(
    _prop_body,
    out_type=jax.ShapeDtypeStruct((NC, NUM_NODES, D), jnp.float32),
    mesh=_mesh,
    scratch_types=(
        [pltpu.VMEM((NB, 1, HC), jnp.int32)] * 4
        + [pltpu.VMEM((HC, D), jnp.float32)] * (4 * NB)
        + [pltpu.VMEM_SHARED((NUM_NODES, D), jnp.float32)]
        + [pltpu.SemaphoreType.DMA] * 2
        + [pltpu.SemaphoreType.DMA] * (4 * NB)
    ),
)

_scale = pl.pallas_call(
    _scale_body,
    out_shape=jax.ShapeDtypeStruct((NUM_NODES, D), jnp.float32),
)

_final = pl.pallas_call(
    _final_body,
    out_shape=jax.ShapeDtypeStruct((NUM_NODES, D), jnp.float32),
)


@jax.jit
def kernel(x, edge_index):
    ei = edge_index.astype(jnp.int32)
    src2 = ei[0].reshape(NUM_EDGES // HC, 1, HC)
    dst2 = ei[1].reshape(NUM_EDGES // HC, 1, HC)
    ep2 = ei.reshape(2 * NUM_EDGES // HCH, 1, HCH)
    deg_part = _hist(ep2)
    degt = deg_part.T
    y = _scale(degt, x)
    p = _prop(y, src2, dst2)
    return _final(degt, p)
